# baseline (speedup 1.0000x reference)
import functools
import math

import jax
import jax.numpy as jnp
from jax import lax
from jax.experimental import pallas as pl
from jax.experimental.pallas import tpu as pltpu
from jax.experimental.pallas import tpu_sc as plsc

_LAT = 721
_LON = 1440
_LEVEL = 8
_N = 1000000

_NW = 32
_CHUNK = 1024
_CPW = 31
_PPW = _CHUNK * _CPW
_NPAD = _PPW * _NW
_GROUPS = _CHUNK // 16


_LONP = 1536


def _interp_matrix(h, H, HP=None):
    HP = H if HP is None else HP
    dsti = lax.broadcasted_iota(jnp.int32, (HP, h), 0)
    dst = dsti.astype(jnp.float32)
    src = lax.broadcasted_iota(jnp.int32, (HP, h), 1)
    ys = jnp.maximum((dst + 0.5) * (h / H) - 0.5, 0.0)
    y0 = jnp.floor(ys).astype(jnp.int32)
    y1 = jnp.minimum(y0 + 1, h - 1)
    fy = ys - y0.astype(jnp.float32)
    w = jnp.where(src == y0, 1.0 - fy, 0.0) + jnp.where(src == y1, fy, 0.0)
    return jnp.where(dsti < H, w, 0.0)


def _upsample_body(g_ref, o_ref, *, h, w):
    g = g_ref[...]
    cxt = _interp_matrix(w, _LON, _LONP).T
    ry = _interp_matrix(h, _LAT)
    m = jnp.dot(g, cxt, preferred_element_type=jnp.float32)
    o_ref[...] = jnp.dot(ry, m, preferred_element_type=jnp.float32)


def _upsample_level(g2d):
    h, w = g2d.shape
    return pl.pallas_call(
        functools.partial(_upsample_body, h=h, w=w),
        out_shape=jax.ShapeDtypeStruct((_LAT, _LONP), jnp.float32),
    )(g2d)


def _interleave_body(*refs):
    us = refs[:_LEVEL]
    o_ref = refs[_LEVEL]
    xb = pl.program_id(0)
    w128 = pl.multiple_of((xb // 8) * 128, 128)
    w128b = pl.multiple_of(jnp.minimum(w128 + 128, _LONP - 128), 128)
    o = (xb % 8) * 16
    xl = lax.broadcasted_iota(jnp.int32, (17, 256), 1) // 16
    l = lax.broadcasted_iota(jnp.int32, (17, 256), 1) % 16
    dxr = lax.broadcasted_iota(jnp.int32, (17, 256), 0)
    shift = (l >= 8).astype(jnp.int32)
    lev = jnp.where(l < 8, l, 15 - l)
    dx = jnp.minimum(xb * 16 + xl + shift, _LON - 1) - xb * 16
    acc = jnp.zeros((_LAT, 256), jnp.float32)
    for i in range(_LEVEL):
        wm = jnp.where((dxr == dx) & (lev == i), 1.0, 0.0)
        wina = us[i][:, pl.ds(w128, 128)]
        rolled = pltpu.roll(wina, -o, axis=1)
        winb = us[i][:, pl.ds(w128b, 128)]
        col17 = jnp.where(o == 112, winb[:, 0:1], rolled[:, 16:17])
        b = jnp.concatenate([rolled[:, :16], col17], axis=1)
        acc = acc + jnp.dot(b, wm, preferred_element_type=jnp.float32)
    o_ref[0] = acc


def _interleave(ups):
    return pl.pallas_call(
        _interleave_body,
        grid=(_LON // 16,),
        in_specs=[pl.BlockSpec((_LAT, _LONP), lambda i: (0, 0))] * _LEVEL,
        out_specs=pl.BlockSpec((1, _LAT, 256), lambda i: (i, 0, 0)),
        out_shape=jax.ShapeDtypeStruct((_LON // 16, _LAT, 256), jnp.float32),
    )(*ups)


def _sc_sample_body(t_hbm, lat_hbm, lon_hbm, out_hbm,
                    latv, lonv, idx0, idx1, wyb, wxb,
                    rows0, rows1, outb, sems):
    nc = 2
    wid = lax.axis_index("s") * nc + lax.axis_index("c")
    base = wid * _PPW

    def stage(ci, buf):
        pbase = base + ci * _CHUNK
        pltpu.sync_copy(lat_hbm.at[pl.ds(pbase, _CHUNK)], latv)
        pltpu.sync_copy(lon_hbm.at[pl.ds(pbase, _CHUNK)], lonv)

        def index_body(g, _):
            la = latv[pl.ds(g * 16, 16)]
            lo = lonv[pl.ds(g * 16, 16)]
            y = (90.0 - la) * 4.0
            x = lo * 4.0
            y0 = jnp.clip(y.astype(jnp.int32), 0, _LAT - 1)
            x0 = jnp.clip(x.astype(jnp.int32), 0, _LON - 1)
            wyb[buf, pl.ds(g * 16, 16)] = y - y0.astype(jnp.float32)
            wxb[buf, pl.ds(g * 16, 16)] = x - x0.astype(jnp.float32)
            y1 = jnp.minimum(y0 + 1, _LAT - 1)
            xblk = (x0 >> 4) * (_LAT * 16) + (x0 & 15)
            idx0[buf, pl.ds(g * 16, 16)] = xblk + y0 * 16
            idx1[buf, pl.ds(g * 16, 16)] = xblk + y1 * 16
            return 0

        lax.fori_loop(0, _GROUPS, index_body, 0, unroll=False)
        for j in range(_CHUNK // 128):
            pltpu.async_copy(
                t_hbm.at[idx0.at[buf, pl.ds(j * 128, 128)]],
                rows0.at[buf, pl.ds(j * 128, 128), :], sems.at[buf])
            pltpu.async_copy(
                t_hbm.at[idx1.at[buf, pl.ds(j * 128, 128)]],
                rows1.at[buf, pl.ds(j * 128, 128), :], sems.at[buf])

    def drain(buf):
        for j in range(_CHUNK // 128):
            pltpu.make_async_copy(
                t_hbm.at[idx0.at[buf, pl.ds(j * 128, 128)]],
                rows0.at[buf, pl.ds(j * 128, 128), :], sems.at[buf]).wait()
            pltpu.make_async_copy(
                t_hbm.at[idx1.at[buf, pl.ds(j * 128, 128)]],
                rows1.at[buf, pl.ds(j * 128, 128), :], sems.at[buf]).wait()

    def chunk_body(ci, _):
        buf = ci % 2

        @pl.when(ci + 1 < _CPW)
        def _():
            stage(ci + 1, (ci + 1) % 2)

        drain(buf)

        def combine_body(g, _):
            wy16 = wyb[buf, pl.ds(g * 16, 16)]
            wx16 = wxb[buf, pl.ds(g * 16, 16)]
            for j in range(16):
                p = g * 16 + j
                va0 = rows0[buf, p, :]
                va1 = rows1[buf, p, :]
                by = jnp.full((16,), wy16[j], jnp.float32)
                bx = jnp.full((16,), wx16[j], jnp.float32)
                m = va0 + by * (va1 - va0)
                mr = lax.rev(m, (0,))
                o = m + bx * (mr - m)
                outb[p, :] = o
            return 0

        lax.fori_loop(0, _GROUPS, combine_body, 0, unroll=False)

        pbase = base + ci * _CHUNK
        pltpu.sync_copy(outb.at[:, pl.ds(0, 8)],
                        out_hbm.at[pl.ds(pbase, _CHUNK), :])
        return 0

    stage(0, 0)
    lax.fori_loop(0, _CPW, chunk_body, 0, unroll=False)


def _sc_sample(table, lat, lon):
    mesh = plsc.VectorSubcoreMesh(core_axis_name="c", subcore_axis_name="s")
    f = pl.kernel(
        _sc_sample_body,
        out_type=jax.ShapeDtypeStruct((_NPAD, _LEVEL), jnp.float32),
        mesh=mesh,
        compiler_params=pltpu.CompilerParams(use_tc_tiling_on_sc=False),
        scratch_types=[
            pltpu.VMEM((_CHUNK,), jnp.float32),
            pltpu.VMEM((_CHUNK,), jnp.float32),
            pltpu.VMEM((2, _CHUNK), jnp.int32),
            pltpu.VMEM((2, _CHUNK), jnp.int32),
            pltpu.VMEM((2, _CHUNK), jnp.float32),
            pltpu.VMEM((2, _CHUNK), jnp.float32),
            pltpu.VMEM((2, _CHUNK, 16), jnp.float32),
            pltpu.VMEM((2, _CHUNK, 16), jnp.float32),
            pltpu.VMEM((_CHUNK, 16), jnp.float32),
            pltpu.SemaphoreType.DMA((2,)),
        ],
    )
    return f(table, lat, lon)


def kernel(x, grid_0, grid_1, grid_2, grid_3, grid_4, grid_5, grid_6, grid_7):
    grids = [grid_0, grid_1, grid_2, grid_3, grid_4, grid_5, grid_6, grid_7]
    ups = [jnp.pad(grids[0][0, 0], ((0, 0), (0, _LONP - _LON)))]
    for g in grids[1:]:
        ups.append(_upsample_level(g[0, 0]))
    table = _interleave(ups).reshape(_LAT * _LON, 16)

    lat = jnp.pad(x[:, 0], (0, _NPAD - _N))
    lon = jnp.pad(x[:, 1], (0, _NPAD - _N))
    return _sc_sample(table, lat, lon)[:_N]
